# baseline (speedup 1.0000x reference)
import dataclasses
import functools

import jax
import jax.numpy as jnp
from jax import lax
from jax.experimental import pallas as pl
from jax.experimental.pallas import tpu as pltpu
from jax.experimental.pallas import tpu_sc as plsc

_NC = 2
_NS = 16
_L = 16
_W = 16384
_PW = 8192
_NB = 6
_KSC = 48


def _tc_detile(utable, itable, ksc, w):
    K, M = utable.shape
    _, N = itable.shape
    kt = K - ksc

    def body(u_ref, i_ref, *out_refs):
        for j in range(K):
            out_refs[j][...] = u_ref[j, :]
        for j in range(kt):
            out_refs[K + j][...] = i_ref[j, :]

    return pl.pallas_call(
        body,
        grid=(pl.cdiv(N, w),),
        in_specs=[
            pl.BlockSpec((K, w), lambda s: (0, s)),
            pl.BlockSpec((kt, w), lambda s, r=ksc // kt: (r, s)),
        ],
        out_specs=([pl.BlockSpec((w,), lambda s: (s,)) for _ in range(K)]
                   + [pl.BlockSpec((w,), lambda s: (s,)) for _ in range(kt)]),
        out_shape=([jax.ShapeDtypeStruct((M,), utable.dtype) for _ in range(K)]
                   + [jax.ShapeDtypeStruct((N,), itable.dtype) for _ in range(kt)]),
    )(utable, itable)


def kernel(user, item, user_emb, item_emb):
    B = user.shape[0]
    K, N = item_emb.shape
    M = user_emb.shape[0]
    NW = _NC * _NS
    CH = B // NW
    KT = K - _KSC

    uT = user_emb.T
    mesh = plsc.VectorSubcoreMesh(core_axis_name="c", subcore_axis_name="s")
    cp = pltpu.CompilerParams()
    if "needs_layout_passes" in pltpu.CompilerParams.__dataclass_fields__:
        cp = dataclasses.replace(cp, needs_layout_passes=False)
    if "use_tc_tiling_on_sc" in pltpu.CompilerParams.__dataclass_fields__:
        cp = dataclasses.replace(cp, use_tc_tiling_on_sc=True)

    nfull = (N // _PW) * _PW
    nchunk = N // _PW
    tail = N - nfull

    @functools.partial(
        pl.kernel,
        out_type=jax.ShapeDtypeStruct((_KSC * N,), jnp.float32),
        mesh=mesh,
        compiler_params=cp,
        scratch_types=[
            pltpu.VMEM((_NB * _PW,), jnp.float32),
            pltpu.VMEM((_NB * _PW,), jnp.float32),
            pltpu.VMEM((tail,), jnp.float32),
            pltpu.SemaphoreType.DMA,
            pltpu.SemaphoreType.DMA,
            pltpu.SemaphoreType.DMA,
        ],
    )
    def sc_detile(tab_hbm, out_hbm, buf_a, buf_b, tail_v, sem_a, sem_b, sem_t):
        wid = lax.axis_index("s") * _NC + lax.axis_index("c")
        nch_half = nchunk // 2
        halves_per_w = (2 * _KSC) // NW

        for t in range(halves_per_w):
            h = wid * halves_per_w + t
            kk = h // 2
            part = h % 2
            c0 = part * nch_half
            nch = nch_half

            def read_batch(b0, nb, buf, sem, kk=kk, c0=c0):
                cps = []
                for q in range(nb):
                    cps.append(pltpu.async_copy(
                        tab_hbm.at[kk].at[pl.ds((c0 + b0 + q) * _PW, _PW)],
                        buf.at[pl.ds(q * _PW, _PW)], sem))
                return cps

            def write_batch(b0, nb, buf, sem, kk=kk, c0=c0):
                cps = []
                for q in range(nb):
                    cps.append(pltpu.async_copy(
                        buf.at[pl.ds(q * _PW, _PW)],
                        out_hbm.at[pl.ds(kk * N + (c0 + b0 + q) * _PW, _PW)],
                        sem))
                return cps

            nbat = nch // _NB
            rem = nch - nbat * _NB
            prev_writes = []
            cur_reads = read_batch(0, _NB, buf_a, sem_a)
            for b in range(nbat):
                nxt = b + 1
                use_a = (b % 2) == 0
                buf = buf_a if use_a else buf_b
                nbuf = buf_b if use_a else buf_a
                nsem = sem_b if use_a else sem_a
                for c in cur_reads:
                    c.wait()
                if nxt < nbat:
                    nxt_reads = read_batch(nxt * _NB, _NB, nbuf, nsem)
                elif rem > 0:
                    nxt_reads = read_batch(nbat * _NB, rem, nbuf, nsem)
                else:
                    nxt_reads = []
                for c in prev_writes:
                    c.wait()
                prev_writes = write_batch(b * _NB, _NB, buf,
                                          sem_a if use_a else sem_b)
                cur_reads = nxt_reads
            if rem > 0:
                buf = buf_a if (nbat % 2) == 0 else buf_b
                for c in cur_reads:
                    c.wait()
                for c in prev_writes:
                    c.wait()
                prev_writes = write_batch(nbat * _NB, rem, buf,
                                          sem_a if (nbat % 2) == 0 else sem_b)
            if tail > 0:
                @pl.when(part == 1)
                def _tail(kk=kk):
                    pltpu.async_copy(
                        tab_hbm.at[kk].at[pl.ds(2 * nch_half * _PW,
                                                N - 2 * nch_half * _PW)],
                        tail_v, sem_t).wait()
                    pltpu.async_copy(
                        tail_v,
                        out_hbm.at[pl.ds(kk * N + 2 * nch_half * _PW,
                                         N - 2 * nch_half * _PW)],
                        sem_t).wait()
            for c in prev_writes:
                c.wait()

    @functools.partial(
        pl.kernel,
        out_type=jax.ShapeDtypeStruct((B,), jnp.float32),
        mesh=mesh,
        compiler_params=cp,
        scratch_types=[
            pltpu.VMEM((CH,), jnp.int32),
            pltpu.VMEM((CH,), jnp.int32),
            pltpu.VMEM((_KSC * CH,), jnp.int32),
            pltpu.VMEM((K * CH,), jnp.float32),
            pltpu.VMEM((K * CH,), jnp.float32),
            pltpu.VMEM((CH,), jnp.float32),
            pltpu.SemaphoreType.DMA,
            pltpu.SemaphoreType.DMA,
        ],
    )
    def sc_gather(*refs):
        user_hbm, item_hbm, iflat_hbm = refs[0], refs[1], refs[2]
        utabs = refs[3:3 + K]
        itabs = refs[3 + K:3 + K + KT]
        out_hbm = refs[3 + K + KT]
        (uidx_v, iidx_v, gidx_v, u_v, v_v, acc_v,
         sem_u, sem_v) = refs[4 + K + KT:12 + K + KT]

        wid = lax.axis_index("s") * _NC + lax.axis_index("c")
        base = wid * CH
        pltpu.sync_copy(user_hbm.at[pl.ds(base, CH)], uidx_v)
        pltpu.sync_copy(item_hbm.at[pl.ds(base, CH)], iidx_v)

        cps = []
        for j in range(K):
            cps.append(pltpu.async_copy(
                utabs[j].at[uidx_v], u_v.at[pl.ds(j * CH, CH)], sem_u))
        for j in range(KT):
            cps.append(pltpu.async_copy(
                itabs[j].at[iidx_v], v_v.at[pl.ds((_KSC + j) * CH, CH)], sem_v))

        @pl.loop(0, _KSC)
        def _build(kk):
            off = kk * N
            for c in range(CH // _L):
                gidx_v[pl.ds(kk * CH + c * _L, _L)] = (
                    iidx_v[pl.ds(c * _L, _L)] + off)

        cp_v = pltpu.async_copy(
            iflat_hbm.at[gidx_v], v_v.at[pl.ds(0, _KSC * CH)], sem_v)
        for c in cps:
            c.wait()
        cp_v.wait()

        @pl.loop(0, CH, step=_L)
        def _mac(c):
            acc = jnp.zeros((_L,), jnp.float32)
            for j in range(K):
                acc = acc + u_v[pl.ds(j * CH + c, _L)] * v_v[pl.ds(j * CH + c, _L)]
            acc = jnp.maximum(4.0 - acc, 0.0)
            acc = jnp.maximum(4.0 - acc, 0.0) + 1.0
            acc_v[pl.ds(c, _L)] = acc

        pltpu.sync_copy(acc_v, out_hbm.at[pl.ds(base, CH)])

    tabs = _tc_detile(uT, item_emb, _KSC, _W)
    utabs, itabs = tabs[:K], tabs[K:]
    iflat = sc_detile(item_emb)
    return sc_gather(user, item, iflat, *utabs, *itabs)


# speedup vs baseline: 1.0187x; 1.0187x over previous
import dataclasses
import functools

import jax
import jax.numpy as jnp
from jax import lax
from jax.experimental import pallas as pl
from jax.experimental.pallas import tpu as pltpu
from jax.experimental.pallas import tpu_sc as plsc

_NC = 2
_NS = 16
_L = 16
_W = 32768
_PW = 8192
_NB = 6
_KSC = 32


def _tc_detile(utable, itable, ksc, w):
    K, M = utable.shape
    _, N = itable.shape
    kt = K - ksc

    def body(u_ref, i_ref, *out_refs):
        for j in range(K):
            out_refs[j][...] = u_ref[j, :]
        for j in range(kt):
            out_refs[K + j][...] = i_ref[j, :]

    return pl.pallas_call(
        body,
        grid=(pl.cdiv(N, w),),
        in_specs=[
            pl.BlockSpec((K, w), lambda s: (0, s)),
            pl.BlockSpec((kt, w), lambda s, r=ksc // kt: (r, s)),
        ],
        out_specs=([pl.BlockSpec((w,), lambda s: (s,)) for _ in range(K)]
                   + [pl.BlockSpec((w,), lambda s: (s,)) for _ in range(kt)]),
        out_shape=([jax.ShapeDtypeStruct((M,), utable.dtype) for _ in range(K)]
                   + [jax.ShapeDtypeStruct((N,), itable.dtype) for _ in range(kt)]),
    )(utable, itable)


def kernel(user, item, user_emb, item_emb):
    B = user.shape[0]
    K, N = item_emb.shape
    M = user_emb.shape[0]
    NW = _NC * _NS
    CH = B // NW
    KT = K - _KSC

    uT = user_emb.T
    mesh = plsc.VectorSubcoreMesh(core_axis_name="c", subcore_axis_name="s")
    cp = pltpu.CompilerParams()
    if "needs_layout_passes" in pltpu.CompilerParams.__dataclass_fields__:
        cp = dataclasses.replace(cp, needs_layout_passes=False)
    if "use_tc_tiling_on_sc" in pltpu.CompilerParams.__dataclass_fields__:
        cp = dataclasses.replace(cp, use_tc_tiling_on_sc=True)

    nfull = (N // _PW) * _PW
    nchunk = N // _PW
    tail = N - nfull

    @functools.partial(
        pl.kernel,
        out_type=jax.ShapeDtypeStruct((_KSC * N,), jnp.float32),
        mesh=mesh,
        compiler_params=cp,
        scratch_types=[
            pltpu.VMEM((_NB * _PW,), jnp.float32),
            pltpu.VMEM((_NB * _PW,), jnp.float32),
            pltpu.VMEM((tail,), jnp.float32),
            pltpu.SemaphoreType.DMA,
            pltpu.SemaphoreType.DMA,
            pltpu.SemaphoreType.DMA,
        ],
    )
    def sc_detile(tab_hbm, out_hbm, buf_a, buf_b, tail_v, sem_a, sem_b, sem_t):
        wid = lax.axis_index("s") * _NC + lax.axis_index("c")
        kk = wid

        def read_batch(b0, nb, buf, sem):
            cps = []
            for q in range(nb):
                cps.append(pltpu.async_copy(
                    tab_hbm.at[kk].at[pl.ds((b0 + q) * _PW, _PW)],
                    buf.at[pl.ds(q * _PW, _PW)], sem))
            return cps

        def write_batch(b0, nb, buf, sem):
            cps = []
            for q in range(nb):
                cps.append(pltpu.async_copy(
                    buf.at[pl.ds(q * _PW, _PW)],
                    out_hbm.at[pl.ds(kk * N + (b0 + q) * _PW, _PW)], sem))
            return cps

        nbat = nchunk // _NB
        rem = nchunk - nbat * _NB
        prev_writes = []
        cur_reads = read_batch(0, _NB, buf_a, sem_a)
        for b in range(nbat):
            nxt = b + 1
            use_a = (b % 2) == 0
            buf = buf_a if use_a else buf_b
            nbuf = buf_b if use_a else buf_a
            nsem = sem_b if use_a else sem_a
            for c in cur_reads:
                c.wait()
            if nxt < nbat:
                nxt_reads = read_batch(nxt * _NB, _NB, nbuf, nsem)
            elif rem > 0:
                nxt_reads = read_batch(nbat * _NB, rem, nbuf, nsem)
            else:
                nxt_reads = []
            for c in prev_writes:
                c.wait()
            prev_writes = write_batch(b * _NB, _NB, buf,
                                      sem_a if use_a else sem_b)
            cur_reads = nxt_reads
        if rem > 0:
            buf = buf_a if (nbat % 2) == 0 else buf_b
            for c in cur_reads:
                c.wait()
            for c in prev_writes:
                c.wait()
            prev_writes = write_batch(nbat * _NB, rem, buf,
                                      sem_a if (nbat % 2) == 0 else sem_b)
        if tail > 0:
            pltpu.async_copy(
                tab_hbm.at[kk].at[pl.ds(nfull, tail)], tail_v, sem_t).wait()
            for c in prev_writes:
                c.wait()
            pltpu.async_copy(
                tail_v, out_hbm.at[pl.ds(kk * N + nfull, tail)], sem_t).wait()
        else:
            for c in prev_writes:
                c.wait()

    @functools.partial(
        pl.kernel,
        out_type=jax.ShapeDtypeStruct((B,), jnp.float32),
        mesh=mesh,
        compiler_params=cp,
        scratch_types=[
            pltpu.VMEM((CH,), jnp.int32),
            pltpu.VMEM((CH,), jnp.int32),
            pltpu.VMEM((_KSC * CH,), jnp.int32),
            pltpu.VMEM((K * CH,), jnp.float32),
            pltpu.VMEM((K * CH,), jnp.float32),
            pltpu.VMEM((CH,), jnp.float32),
            pltpu.SemaphoreType.DMA,
            pltpu.SemaphoreType.DMA,
        ],
    )
    def sc_gather(*refs):
        user_hbm, item_hbm, iflat_hbm = refs[0], refs[1], refs[2]
        utabs = refs[3:3 + K]
        itabs = refs[3 + K:3 + K + KT]
        out_hbm = refs[3 + K + KT]
        (uidx_v, iidx_v, gidx_v, u_v, v_v, acc_v,
         sem_u, sem_v) = refs[4 + K + KT:12 + K + KT]

        wid = lax.axis_index("s") * _NC + lax.axis_index("c")
        base = wid * CH
        pltpu.sync_copy(user_hbm.at[pl.ds(base, CH)], uidx_v)
        pltpu.sync_copy(item_hbm.at[pl.ds(base, CH)], iidx_v)

        cps = []
        for j in range(K):
            cps.append(pltpu.async_copy(
                utabs[j].at[uidx_v], u_v.at[pl.ds(j * CH, CH)], sem_u))
        for j in range(KT):
            cps.append(pltpu.async_copy(
                itabs[j].at[iidx_v], v_v.at[pl.ds((_KSC + j) * CH, CH)], sem_v))

        @pl.loop(0, _KSC)
        def _build(kk):
            off = kk * N
            for c in range(CH // _L):
                gidx_v[pl.ds(kk * CH + c * _L, _L)] = (
                    iidx_v[pl.ds(c * _L, _L)] + off)

        cp_v = pltpu.async_copy(
            iflat_hbm.at[gidx_v], v_v.at[pl.ds(0, _KSC * CH)], sem_v)
        for c in cps:
            c.wait()
        cp_v.wait()

        @pl.loop(0, CH, step=_L)
        def _mac(c):
            acc = jnp.zeros((_L,), jnp.float32)
            for j in range(K):
                acc = acc + u_v[pl.ds(j * CH + c, _L)] * v_v[pl.ds(j * CH + c, _L)]
            acc = jnp.maximum(4.0 - acc, 0.0)
            acc = jnp.maximum(4.0 - acc, 0.0) + 1.0
            acc_v[pl.ds(c, _L)] = acc

        pltpu.sync_copy(acc_v, out_hbm.at[pl.ds(base, CH)])

    tabs = _tc_detile(uT, item_emb, _KSC, _W)
    utabs, itabs = tabs[:K], tabs[K:]
    iflat = sc_detile(item_emb)
    return sc_gather(user, item, iflat, *utabs, *itabs)


# speedup vs baseline: 1.0203x; 1.0016x over previous
import dataclasses
import functools

import jax
import jax.numpy as jnp
from jax import lax
from jax.experimental import pallas as pl
from jax.experimental.pallas import tpu as pltpu
from jax.experimental.pallas import tpu_sc as plsc

_NC = 2
_NS = 16
_L = 16
_W = 32768
_PW = 8192
_NB = 6
_KSC = 32


def _tc_detile(utable, itable, ksc, w):
    K, M = utable.shape
    _, N = itable.shape
    kt = K - ksc

    def body(u_ref, i_ref, *out_refs):
        for j in range(K):
            out_refs[j][...] = u_ref[j, :]
        for j in range(kt):
            out_refs[K + j][...] = i_ref[j, :]

    return pl.pallas_call(
        body,
        grid=(pl.cdiv(N, w),),
        in_specs=[
            pl.BlockSpec((K, w), lambda s: (0, s)),
            pl.BlockSpec((kt, w), lambda s, r=ksc // kt: (r, s)),
        ],
        out_specs=([pl.BlockSpec((w,), lambda s: (s,)) for _ in range(K)]
                   + [pl.BlockSpec((w,), lambda s: (s,)) for _ in range(kt)]),
        out_shape=([jax.ShapeDtypeStruct((M,), utable.dtype) for _ in range(K)]
                   + [jax.ShapeDtypeStruct((N,), itable.dtype) for _ in range(kt)]),
    )(utable, itable)


def kernel(user, item, user_emb, item_emb):
    B = user.shape[0]
    K, N = item_emb.shape
    M = user_emb.shape[0]
    NW = _NC * _NS
    CH = B // NW
    KT = K - _KSC

    uT = user_emb.T
    mesh = plsc.VectorSubcoreMesh(core_axis_name="c", subcore_axis_name="s")
    cp = pltpu.CompilerParams()
    if "needs_layout_passes" in pltpu.CompilerParams.__dataclass_fields__:
        cp = dataclasses.replace(cp, needs_layout_passes=False)
    if "use_tc_tiling_on_sc" in pltpu.CompilerParams.__dataclass_fields__:
        cp = dataclasses.replace(cp, use_tc_tiling_on_sc=True)

    nfull = (N // _PW) * _PW
    nchunk = N // _PW
    tail = N - nfull

    @functools.partial(
        pl.kernel,
        out_type=jax.ShapeDtypeStruct((_KSC * N,), jnp.float32),
        mesh=mesh,
        compiler_params=cp,
        scratch_types=[
            pltpu.VMEM((_NB * _PW,), jnp.float32),
            pltpu.VMEM((_NB * _PW,), jnp.float32),
            pltpu.VMEM((tail,), jnp.float32),
            pltpu.SemaphoreType.DMA,
            pltpu.SemaphoreType.DMA,
            pltpu.SemaphoreType.DMA,
        ],
    )
    def sc_detile(tab_hbm, out_hbm, buf_a, buf_b, tail_v, sem_a, sem_b, sem_t):
        wid = lax.axis_index("s") * _NC + lax.axis_index("c")
        kk = wid

        def read_batch(b0, nb, buf, sem):
            cps = []
            for q in range(nb):
                cps.append(pltpu.async_copy(
                    tab_hbm.at[kk].at[pl.ds((b0 + q) * _PW, _PW)],
                    buf.at[pl.ds(q * _PW, _PW)], sem))
            return cps

        def write_batch(b0, nb, buf, sem):
            cps = []
            for q in range(nb):
                cps.append(pltpu.async_copy(
                    buf.at[pl.ds(q * _PW, _PW)],
                    out_hbm.at[pl.ds(kk * N + (b0 + q) * _PW, _PW)], sem))
            return cps

        nbat = nchunk // _NB
        rem = nchunk - nbat * _NB
        prev_writes = []
        cur_reads = read_batch(0, _NB, buf_a, sem_a)
        for b in range(nbat):
            nxt = b + 1
            use_a = (b % 2) == 0
            buf = buf_a if use_a else buf_b
            nbuf = buf_b if use_a else buf_a
            nsem = sem_b if use_a else sem_a
            for c in cur_reads:
                c.wait()
            if nxt < nbat:
                nxt_reads = read_batch(nxt * _NB, _NB, nbuf, nsem)
            elif rem > 0:
                nxt_reads = read_batch(nbat * _NB, rem, nbuf, nsem)
            else:
                nxt_reads = []
            for c in prev_writes:
                c.wait()
            prev_writes = write_batch(b * _NB, _NB, buf,
                                      sem_a if use_a else sem_b)
            cur_reads = nxt_reads
        if rem > 0:
            buf = buf_a if (nbat % 2) == 0 else buf_b
            for c in cur_reads:
                c.wait()
            for c in prev_writes:
                c.wait()
            prev_writes = write_batch(nbat * _NB, rem, buf,
                                      sem_a if (nbat % 2) == 0 else sem_b)
        if tail > 0:
            pltpu.async_copy(
                tab_hbm.at[kk].at[pl.ds(nfull, tail)], tail_v, sem_t).wait()
            for c in prev_writes:
                c.wait()
            pltpu.async_copy(
                tail_v, out_hbm.at[pl.ds(kk * N + nfull, tail)], sem_t).wait()
        else:
            for c in prev_writes:
                c.wait()

    @functools.partial(
        pl.kernel,
        out_type=jax.ShapeDtypeStruct((B,), jnp.float32),
        mesh=mesh,
        compiler_params=cp,
        scratch_types=[
            pltpu.VMEM((CH,), jnp.int32),
            pltpu.VMEM((CH,), jnp.int32),
            pltpu.VMEM((K * CH,), jnp.float32),
            pltpu.VMEM((K * CH,), jnp.float32),
            pltpu.VMEM((CH,), jnp.float32),
            pltpu.SemaphoreType.DMA,
            pltpu.SemaphoreType.DMA,
        ],
    )
    def sc_gather(*refs):
        user_hbm, item_hbm, iflat_hbm = refs[0], refs[1], refs[2]
        utabs = refs[3:3 + K]
        itabs = refs[3 + K:3 + K + KT]
        out_hbm = refs[3 + K + KT]
        (uidx_v, iidx_v, u_v, v_v, acc_v,
         sem_u, sem_v) = refs[4 + K + KT:11 + K + KT]

        wid = lax.axis_index("s") * _NC + lax.axis_index("c")
        base = wid * CH
        pltpu.sync_copy(user_hbm.at[pl.ds(base, CH)], uidx_v)
        pltpu.sync_copy(item_hbm.at[pl.ds(base, CH)], iidx_v)

        cps = []
        for j in range(K):
            cps.append(pltpu.async_copy(
                utabs[j].at[uidx_v], u_v.at[pl.ds(j * CH, CH)], sem_u))
        for j in range(KT):
            cps.append(pltpu.async_copy(
                itabs[j].at[iidx_v], v_v.at[pl.ds((_KSC + j) * CH, CH)], sem_v))
        for j in range(_KSC):
            cps.append(pltpu.async_copy(
                iflat_hbm.at[pl.ds(j * N, N)].at[iidx_v],
                v_v.at[pl.ds(j * CH, CH)], sem_v))
        for c in cps:
            c.wait()

        @pl.loop(0, CH, step=_L)
        def _mac(c):
            acc = jnp.zeros((_L,), jnp.float32)
            for j in range(K):
                acc = acc + u_v[pl.ds(j * CH + c, _L)] * v_v[pl.ds(j * CH + c, _L)]
            acc = jnp.maximum(4.0 - acc, 0.0)
            acc = jnp.maximum(4.0 - acc, 0.0) + 1.0
            acc_v[pl.ds(c, _L)] = acc

        pltpu.sync_copy(acc_v, out_hbm.at[pl.ds(base, CH)])

    tabs = _tc_detile(uT, item_emb, _KSC, _W)
    utabs, itabs = tabs[:K], tabs[K:]
    iflat = sc_detile(item_emb)
    return sc_gather(user, item, iflat, *utabs, *itabs)
